# Initial kernel scaffold; baseline (speedup 1.0000x reference)
import jax
import jax.numpy as jnp
from jax.experimental import pallas as pl


def kernel(query, queue):
    raise NotImplementedError("write your pallas kernel here")


# speedup vs baseline: 77.7354x; 77.7354x over previous
import functools

import jax
import jax.numpy as jnp
from jax.experimental import pallas as pl
from jax.experimental.pallas import tpu as pltpu
from jax.experimental.pallas import tpu_sc as plsc

_DIM = 64
_QUEUE = 100000
_K = 5
_TEMP = 0.1
_BATCH = 1024

_BLK = 2048
_NBLK = (_QUEUE + _BLK - 1) // _BLK
_PADQ = _NBLK * _BLK

_NEG = float("-inf")
_BIGI = 2**30


def _extract_topk(vals, idxs, k):
    out_v, out_i = [], []
    for _ in range(k):
        m = jnp.max(vals, axis=1, keepdims=True)
        cand = jnp.where(vals == m, idxs, _BIGI)
        ai = jnp.min(cand, axis=1, keepdims=True)
        vals = jnp.where(idxs == ai, _NEG, vals)
        out_v.append(m)
        out_i.append(ai)
    return out_v, out_i


def _topk_kernel(q_ref, qb_ref, w_ref, i_ref, d_ref, rv_ref, ri_ref):
    blk = pl.program_id(0)

    @pl.when(blk == 0)
    def _init():
        rv_ref[...] = jnp.full((_BATCH, 8), _NEG, jnp.float32)
        ri_ref[...] = jnp.full((_BATCH, 8), -1, jnp.int32)

    q = q_ref[...]
    nrm = jnp.sqrt(jnp.sum(q * q, axis=1, keepdims=True))
    qn = q / jnp.maximum(nrm, 1e-12)

    sims = jax.lax.dot_general(
        qn, qb_ref[...], (((1,), (1,)), ((), ())),
        preferred_element_type=jnp.float32)
    col = jax.lax.broadcasted_iota(jnp.int32, (_BATCH, _BLK), 1) + blk * _BLK
    sims = jnp.where(col < _QUEUE, sims, _NEG)

    bv, bi = _extract_topk(sims, col, _K)

    cand_v = jnp.concatenate([rv_ref[...]] + bv, axis=1)
    cand_i = jnp.concatenate([ri_ref[...]] + bi, axis=1)
    nv, ni = _extract_topk(cand_v, cand_i, _K)

    pad_v = [jnp.full((_BATCH, 1), _NEG, jnp.float32)] * 3
    pad_i = [jnp.full((_BATCH, 1), -1, jnp.int32)] * 3
    new_rv = jnp.concatenate(nv + pad_v, axis=1)
    new_ri = jnp.concatenate(ni + pad_i, axis=1)
    rv_ref[...] = new_rv
    ri_ref[...] = new_ri

    @pl.when(blk == _NBLK - 1)
    def _final():
        v = new_rv[:, :_K]
        top = new_rv[:, 0:1]
        e = jnp.exp((v - top) / _TEMP)
        w = e / jnp.sum(e, axis=1, keepdims=True)
        w_ref[...] = jnp.concatenate(
            [w, jnp.zeros((_BATCH, 3), jnp.float32)], axis=1)
        i_ref[...] = new_ri
        d_ref[...] = jnp.broadcast_to(1.0 - top, (_BATCH, 8))


def _run_topk(query, queue_pad):
    out = pl.pallas_call(
        _topk_kernel,
        grid=(_NBLK,),
        in_specs=[
            pl.BlockSpec((_BATCH, _DIM), lambda i: (0, 0)),
            pl.BlockSpec((_BLK, _DIM), lambda i: (i, 0)),
        ],
        out_specs=[
            pl.BlockSpec((_BATCH, 8), lambda i: (0, 0)),
            pl.BlockSpec((_BATCH, 8), lambda i: (0, 0)),
            pl.BlockSpec((_BATCH, 8), lambda i: (0, 0)),
        ],
        out_shape=[
            jax.ShapeDtypeStruct((_BATCH, 8), jnp.float32),
            jax.ShapeDtypeStruct((_BATCH, 8), jnp.int32),
            jax.ShapeDtypeStruct((_BATCH, 8), jnp.float32),
        ],
        scratch_shapes=[
            pltpu.VMEM((_BATCH, 8), jnp.float32),
            pltpu.VMEM((_BATCH, 8), jnp.int32),
        ],
        compiler_params=pltpu.CompilerParams(
            dimension_semantics=("arbitrary",)),
    )(query, queue_pad)
    return out


_GATHER_WIN = 128


def _sc_gather(queue, idx_flat):
    n = idx_flat.shape[0]
    width = queue.shape[1]
    mesh = plsc.VectorSubcoreMesh(core_axis_name="core",
                                  subcore_axis_name="subcore")

    @pl.kernel(
        out_type=jax.ShapeDtypeStruct((n, width), queue.dtype),
        mesh=mesh,
    )
    def kern(x_hbm, i_hbm, o_hbm):
        def body(i_vmem, o_vmem):
            pltpu.sync_copy(x_hbm.at[i_vmem.at[0]], o_vmem)

        pltpu.emit_pipeline(
            body,
            grid=(n // _GATHER_WIN,),
            in_specs=[pl.BlockSpec((1, _GATHER_WIN), index_map=lambda i: (0, i))],
            out_specs=[pl.BlockSpec((_GATHER_WIN, width),
                                    index_map=lambda i: (i, 0))],
            core_axis_name=("core", "subcore"),
            dimension_semantics=(pltpu.PARALLEL,),
        )(i_hbm, o_hbm)

    return kern(queue, idx_flat.reshape(1, n))


def _combine_kernel(g_ref, w_ref, p_ref, o_ref):
    w = w_ref[...]
    p = p_ref[...]
    g = g_ref[...]
    acc = jnp.zeros((_BATCH, _DIM), jnp.float32)
    for j in range(_K):
        pair = g[:, j * 128:(j + 1) * 128]
        row = jnp.where(p[:, j:j + 1] > 0, pair[:, _DIM:], pair[:, :_DIM])
        acc = acc + w[:, j:j + 1] * row
    nrm = jnp.sqrt(jnp.sum(acc * acc, axis=1, keepdims=True))
    o_ref[...] = acc / jnp.maximum(nrm, 1e-12)


def _run_combine(gathered, weights, parity):
    return pl.pallas_call(
        _combine_kernel,
        out_shape=jax.ShapeDtypeStruct((_BATCH, _DIM), jnp.float32),
    )(gathered, weights, parity)


@jax.jit
def kernel(query, queue):
    queue_pad = jnp.pad(queue, ((0, _PADQ - _QUEUE), (0, 0)))
    w8, i8, d8 = _run_topk(query, queue_pad)
    weights = w8[:, :_K]
    idx = i8[:, :_K]
    distances = d8[:, 0]
    queue2 = queue.reshape(_QUEUE // 2, 2 * _DIM)
    parity = jnp.pad(idx & 1, ((0, 0), (0, 3)))
    gathered = _sc_gather(queue2, (idx >> 1).reshape(-1))
    retrieved = _run_combine(gathered.reshape(_BATCH, _K * 2 * _DIM),
                             w8, parity)
    return retrieved, distances, weights


# speedup vs baseline: 136.1565x; 1.7515x over previous
import jax
import jax.numpy as jnp
from jax.experimental import pallas as pl
from jax.experimental.pallas import tpu as pltpu
from jax.experimental.pallas import tpu_sc as plsc

_DIM = 64
_QUEUE = 100000
_K = 5
_TEMP = 0.1
_BATCH = 1024

_BLK = 2048
_NBLK = (_QUEUE + _BLK - 1) // _BLK
_PADQ = _NBLK * _BLK
_NCHUNK = _BLK // 128
_GRP = 8
_GPB = _NCHUNK // _GRP
_NCAND = _K * _GRP

_NEG = float("-inf")
_BIGI = 2**30


def _norm_rows(x):
    n = jnp.sqrt(jnp.sum(x * x, axis=1, keepdims=True))
    return x / jnp.maximum(n, 1e-12)


def _extract_topk(vals, idxs, k):
    out_v, out_i = [], []
    for _ in range(k):
        m = jnp.max(vals, axis=1, keepdims=True)
        cand = jnp.where(vals == m, idxs, _BIGI)
        ai = jnp.min(cand, axis=1, keepdims=True)
        vals = jnp.where(idxs == ai, _NEG, vals)
        out_v.append(m)
        out_i.append(ai)
    return out_v, out_i


def _group_select_kernel(q_ref, qb_ref, i_ref, rv_ref, ri_ref):
    blk = pl.program_id(0)

    @pl.when(blk == 0)
    def _init():
        rv_ref[...] = jnp.full((_BATCH, 8), _NEG, jnp.float32)
        ri_ref[...] = jnp.full((_BATCH, 8), -1, jnp.int32)

    qn = _norm_rows(q_ref[...])
    sims = jax.lax.dot_general(
        qn, qb_ref[...], (((1,), (1,)), ((), ())),
        preferred_element_type=jnp.float32)

    def merge(sims):
        lane = jax.lax.broadcasted_iota(jnp.int32, (_BATCH, 128), 1)
        fv, fi = [rv_ref[...]], [ri_ref[...]]
        for g in range(_GPB):
            f = sims[:, g * _GRP * 128:(g * _GRP + 1) * 128]
            for t in range(1, _GRP):
                c0 = (g * _GRP + t) * 128
                f = jnp.maximum(f, sims[:, c0:c0 + 128])
            fv.append(f)
            fi.append(lane + (blk * _GPB + g) * 128)
        nv, ni = _extract_topk(jnp.concatenate(fv, axis=1),
                               jnp.concatenate(fi, axis=1), _K)
        pad_v = [jnp.full((_BATCH, 1), _NEG, jnp.float32)] * 3
        pad_i = [jnp.full((_BATCH, 1), -1, jnp.int32)] * 3
        rv_ref[...] = jnp.concatenate(nv + pad_v, axis=1)
        new_ri = jnp.concatenate(ni + pad_i, axis=1)
        ri_ref[...] = new_ri

        @pl.when(blk == _NBLK - 1)
        def _final():
            i_ref[...] = new_ri

    @pl.when(blk < _NBLK - 1)
    def _clean():
        merge(sims)

    @pl.when(blk == _NBLK - 1)
    def _tail():
        col = jax.lax.broadcasted_iota(jnp.int32, (_BATCH, _BLK), 1)
        merge(jnp.where(col + blk * _BLK < _QUEUE, sims, _NEG))


def _run_group_select(query, queue_pad):
    return pl.pallas_call(
        _group_select_kernel,
        grid=(_NBLK,),
        in_specs=[
            pl.BlockSpec((_BATCH, _DIM), lambda i: (0, 0)),
            pl.BlockSpec((_BLK, _DIM), lambda i: (i, 0)),
        ],
        out_specs=pl.BlockSpec((_BATCH, 8), lambda i: (0, 0)),
        out_shape=jax.ShapeDtypeStruct((_BATCH, 8), jnp.int32),
        scratch_shapes=[
            pltpu.VMEM((_BATCH, 8), jnp.float32),
            pltpu.VMEM((_BATCH, 8), jnp.int32),
        ],
        compiler_params=pltpu.CompilerParams(
            dimension_semantics=("arbitrary",)),
    )(query, queue_pad)


_GATHER_WIN = 128


def _sc_gather(queue, idx_flat):
    n = idx_flat.shape[0]
    width = queue.shape[1]
    mesh = plsc.VectorSubcoreMesh(core_axis_name="core",
                                  subcore_axis_name="subcore")

    @pl.kernel(
        out_type=jax.ShapeDtypeStruct((n, width), queue.dtype),
        mesh=mesh,
    )
    def kern(x_hbm, i_hbm, o_hbm):
        def body(i_vmem, o_vmem):
            pltpu.sync_copy(x_hbm.at[i_vmem.at[0]], o_vmem)

        pltpu.emit_pipeline(
            body,
            grid=(n // _GATHER_WIN,),
            in_specs=[pl.BlockSpec((1, _GATHER_WIN), index_map=lambda i: (0, i))],
            out_specs=[pl.BlockSpec((_GATHER_WIN, width),
                                    index_map=lambda i: (i, 0))],
            core_axis_name=("core", "subcore"),
            dimension_semantics=(pltpu.PARALLEL,),
        )(i_hbm, o_hbm)

    return kern(queue, idx_flat.reshape(1, n))


_RROWS = 256


def _rescore_kernel(q_ref, g_ref, c_ref, p_ref, o_ref, w_ref, d_ref):
    qn = _norm_rows(q_ref[...])
    cols = c_ref[...]
    par = p_ref[...]

    def vec(j):
        pair = g_ref[:, j * 128:(j + 1) * 128]
        return jnp.where(par[:, j:j + 1] > 0, pair[:, _DIM:], pair[:, :_DIM])

    sims = []
    for j in range(_NCAND):
        sims.append(jnp.sum(qn * vec(j), axis=1, keepdims=True))
    sims = jnp.concatenate(sims, axis=1)
    sims = jnp.where(cols < _QUEUE, sims, _NEG)

    nv, ni = _extract_topk(sims, cols, _K)
    v5 = jnp.concatenate(nv, axis=1)
    top = v5[:, 0:1]
    e = jnp.exp((v5 - top) / _TEMP)
    w5 = e / jnp.sum(e, axis=1, keepdims=True)

    wt = jnp.zeros((_RROWS, _NCAND), jnp.float32)
    for i in range(_K):
        wt = jnp.where(cols == ni[i], w5[:, i:i + 1], wt)

    acc = jnp.zeros((_RROWS, _DIM), jnp.float32)
    for j in range(_NCAND):
        acc = acc + wt[:, j:j + 1] * vec(j)

    o_ref[...] = _norm_rows(acc)
    w_ref[...] = jnp.concatenate(
        [w5, jnp.zeros((_RROWS, 3), jnp.float32)], axis=1)
    d_ref[...] = jnp.broadcast_to(1.0 - top, (_RROWS, 8))


def _run_rescore(query, gathered, cols, par):
    nrow = _BATCH // _RROWS
    return pl.pallas_call(
        _rescore_kernel,
        grid=(nrow,),
        in_specs=[
            pl.BlockSpec((_RROWS, _DIM), lambda i: (i, 0)),
            pl.BlockSpec((_RROWS, _NCAND * 2 * _DIM), lambda i: (i, 0)),
            pl.BlockSpec((_RROWS, _NCAND), lambda i: (i, 0)),
            pl.BlockSpec((_RROWS, _NCAND), lambda i: (i, 0)),
        ],
        out_specs=[
            pl.BlockSpec((_RROWS, _DIM), lambda i: (i, 0)),
            pl.BlockSpec((_RROWS, 8), lambda i: (i, 0)),
            pl.BlockSpec((_RROWS, 8), lambda i: (i, 0)),
        ],
        out_shape=[
            jax.ShapeDtypeStruct((_BATCH, _DIM), jnp.float32),
            jax.ShapeDtypeStruct((_BATCH, 8), jnp.float32),
            jax.ShapeDtypeStruct((_BATCH, 8), jnp.float32),
        ],
        compiler_params=pltpu.CompilerParams(
            dimension_semantics=("arbitrary",)),
    )(query, gathered, cols, par)


@jax.jit
def kernel(query, queue):
    queue_pad = jnp.pad(queue, ((0, _PADQ - _QUEUE), (0, 0)))
    gid8 = _run_group_select(query, queue_pad)
    gid = gid8[:, :_K]

    base = ((gid >> 8) * _BLK) + ((gid >> 7) & 1) * (_GRP * 128) + (gid & 127)
    t = jnp.arange(_GRP, dtype=jnp.int32) * 128
    cols = (base[:, :, None] + t[None, None, :]).reshape(_BATCH, _NCAND)
    safe = jnp.where(cols < _QUEUE, cols, 0)
    par = safe & 1

    queue2 = queue.reshape(_QUEUE // 2, 2 * _DIM)
    gathered = _sc_gather(queue2, (safe >> 1).reshape(-1))
    retrieved, w8, d8 = _run_rescore(
        query, gathered.reshape(_BATCH, _NCAND * 2 * _DIM), cols, par)
    return retrieved, d8[:, 0], w8[:, :_K]


# speedup vs baseline: 199.8483x; 1.4678x over previous
import jax
import jax.numpy as jnp
from jax.experimental import pallas as pl
from jax.experimental.pallas import tpu as pltpu
from jax.experimental.pallas import tpu_sc as plsc

_DIM = 64
_QUEUE = 100000
_K = 5
_TEMP = 0.1
_BATCH = 1024

_BLK = 2048
_NBLK = (_QUEUE + _BLK - 1) // _BLK
_PADQ = _NBLK * _BLK
_NCHUNK = _BLK // 128
_GRP = 8
_GPB = _NCHUNK // _GRP
_NCAND = _K * _GRP

_NEG = float("-inf")
_BIGI = 2**30


def _norm_rows(x):
    n = jnp.sqrt(jnp.sum(x * x, axis=1, keepdims=True))
    return x / jnp.maximum(n, 1e-12)


def _extract_topk(vals, idxs, k):
    out_v, out_i = [], []
    for _ in range(k):
        m = jnp.max(vals, axis=1, keepdims=True)
        cand = jnp.where(vals == m, idxs, _BIGI)
        ai = jnp.min(cand, axis=1, keepdims=True)
        vals = jnp.where(idxs == ai, _NEG, vals)
        out_v.append(m)
        out_i.append(ai)
    return out_v, out_i


_TAIL0 = _QUEUE - _BLK
_TSKIP = (_NBLK - 1) * _BLK - _TAIL0


def _group_select_kernel(q_ref, qb_ref, qt_ref, i_ref, rv_ref, ri_ref):
    blk = pl.program_id(0)

    @pl.when(blk == 0)
    def _init():
        rv_ref[...] = jnp.full((_BATCH, _K * 128), _NEG, jnp.float32)
        ri_ref[...] = jnp.full((_BATCH, _K * 128), -1, jnp.int32)

    qn = _norm_rows(q_ref[...])

    def merge(sims):
        lane = jax.lax.broadcasted_iota(jnp.int32, (_BATCH, 128), 1)
        for g in range(_GPB):
            f = sims[:, g * _GRP * 128:(g * _GRP + 1) * 128]
            for t in range(1, _GRP):
                c0 = (g * _GRP + t) * 128
                f = jnp.maximum(f, sims[:, c0:c0 + 128])
            c = f
            gc = lane + (blk * _GPB + g) * 128
            for i in range(_K):
                s = pl.ds(i * 128, 128)
                r = rv_ref[:, s]
                gr = ri_ref[:, s]
                sel = c > r
                rv_ref[:, s] = jnp.where(sel, c, r)
                ri_ref[:, s] = jnp.where(sel, gc, gr)
                c = jnp.where(sel, r, c)
                gc = jnp.where(sel, gr, gc)

    @pl.when(blk < _NBLK - 1)
    def _clean():
        merge(jax.lax.dot_general(
            qn, qb_ref[...], (((1,), (1,)), ((), ())),
            preferred_element_type=jnp.float32))

    @pl.when(blk == _NBLK - 1)
    def _tail():
        sims = jax.lax.dot_general(
            qn, qt_ref[...], (((1,), (1,)), ((), ())),
            preferred_element_type=jnp.float32)
        col = jax.lax.broadcasted_iota(jnp.int32, (_BATCH, _BLK), 1)
        merge(jnp.where(col >= _TSKIP, sims, _NEG))
        ni = _extract_topk(rv_ref[...], ri_ref[...], _K)[1]
        pad_i = [jnp.full((_BATCH, 1), -1, jnp.int32)] * 3
        i_ref[...] = jnp.concatenate(ni + pad_i, axis=1)


def _run_group_select(query, queue, queue_tail):
    return pl.pallas_call(
        _group_select_kernel,
        grid=(_NBLK,),
        in_specs=[
            pl.BlockSpec((_BATCH, _DIM), lambda i: (0, 0)),
            pl.BlockSpec((_BLK, _DIM),
                         lambda i: (jnp.minimum(i, _NBLK - 2), 0)),
            pl.BlockSpec((_BLK, _DIM), lambda i: (0, 0)),
        ],
        out_specs=pl.BlockSpec((_BATCH, 8), lambda i: (0, 0)),
        out_shape=jax.ShapeDtypeStruct((_BATCH, 8), jnp.int32),
        scratch_shapes=[
            pltpu.VMEM((_BATCH, _K * 128), jnp.float32),
            pltpu.VMEM((_BATCH, _K * 128), jnp.int32),
        ],
        compiler_params=pltpu.CompilerParams(
            dimension_semantics=("arbitrary",)),
    )(query, queue, queue_tail)


_GATHER_WIN = 128


def _sc_gather(queue, idx_flat):
    n = idx_flat.shape[0]
    width = queue.shape[1]
    mesh = plsc.VectorSubcoreMesh(core_axis_name="core",
                                  subcore_axis_name="subcore")

    @pl.kernel(
        out_type=jax.ShapeDtypeStruct((n, width), queue.dtype),
        mesh=mesh,
    )
    def kern(x_hbm, i_hbm, o_hbm):
        def body(i_vmem, o_vmem):
            pltpu.sync_copy(x_hbm.at[i_vmem.at[0]], o_vmem)

        pltpu.emit_pipeline(
            body,
            grid=(n // _GATHER_WIN,),
            in_specs=[pl.BlockSpec((1, _GATHER_WIN), index_map=lambda i: (0, i))],
            out_specs=[pl.BlockSpec((_GATHER_WIN, width),
                                    index_map=lambda i: (i, 0))],
            core_axis_name=("core", "subcore"),
            dimension_semantics=(pltpu.PARALLEL,),
        )(i_hbm, o_hbm)

    return kern(queue, idx_flat.reshape(1, n))


_RROWS = 512


def _rescore_kernel(q_ref, g_ref, c_ref, p_ref, o_ref, w_ref, d_ref):
    qn = _norm_rows(q_ref[...])
    cols = c_ref[...]
    par = p_ref[...]

    def vec(j):
        pair = g_ref[:, j * 128:(j + 1) * 128]
        return jnp.where(par[:, j:j + 1] > 0, pair[:, _DIM:], pair[:, :_DIM])

    sims = []
    for j in range(_NCAND):
        sims.append(jnp.sum(qn * vec(j), axis=1, keepdims=True))
    sims = jnp.concatenate(sims, axis=1)
    sims = jnp.where(cols < _QUEUE, sims, _NEG)

    nv, ni = _extract_topk(sims, cols, _K)
    v5 = jnp.concatenate(nv, axis=1)
    top = v5[:, 0:1]
    e = jnp.exp((v5 - top) / _TEMP)
    w5 = e / jnp.sum(e, axis=1, keepdims=True)

    wt = jnp.zeros((_RROWS, _NCAND), jnp.float32)
    for i in range(_K):
        wt = jnp.where(cols == ni[i], w5[:, i:i + 1], wt)

    acc = jnp.zeros((_RROWS, _DIM), jnp.float32)
    for j in range(_NCAND):
        acc = acc + wt[:, j:j + 1] * vec(j)

    o_ref[...] = _norm_rows(acc)
    w_ref[...] = jnp.concatenate(
        [w5, jnp.zeros((_RROWS, 3), jnp.float32)], axis=1)
    d_ref[...] = jnp.broadcast_to(1.0 - top, (_RROWS, 8))


def _run_rescore(query, gathered, cols, par):
    nrow = _BATCH // _RROWS
    return pl.pallas_call(
        _rescore_kernel,
        grid=(nrow,),
        in_specs=[
            pl.BlockSpec((_RROWS, _DIM), lambda i: (i, 0)),
            pl.BlockSpec((_RROWS, _NCAND * 2 * _DIM), lambda i: (i, 0)),
            pl.BlockSpec((_RROWS, _NCAND), lambda i: (i, 0)),
            pl.BlockSpec((_RROWS, _NCAND), lambda i: (i, 0)),
        ],
        out_specs=[
            pl.BlockSpec((_RROWS, _DIM), lambda i: (i, 0)),
            pl.BlockSpec((_RROWS, 8), lambda i: (i, 0)),
            pl.BlockSpec((_RROWS, 8), lambda i: (i, 0)),
        ],
        out_shape=[
            jax.ShapeDtypeStruct((_BATCH, _DIM), jnp.float32),
            jax.ShapeDtypeStruct((_BATCH, 8), jnp.float32),
            jax.ShapeDtypeStruct((_BATCH, 8), jnp.float32),
        ],
        compiler_params=pltpu.CompilerParams(
            dimension_semantics=("arbitrary",)),
    )(query, gathered, cols, par)


@jax.jit
def kernel(query, queue):
    gid8 = _run_group_select(query, queue, queue[_TAIL0:])
    gid = gid8[:, :_K]

    blk = gid >> 8
    base = (blk * _BLK) + ((gid >> 7) & 1) * (_GRP * 128) + (gid & 127)
    base = base - jnp.where(blk == _NBLK - 1, _TSKIP, 0)
    t = jnp.arange(_GRP, dtype=jnp.int32) * 128
    cols = (base[:, :, None] + t[None, None, :]).reshape(_BATCH, _NCAND)
    safe = jnp.where(cols < _QUEUE, cols, 0)
    par = safe & 1

    queue2 = queue.reshape(_QUEUE // 2, 2 * _DIM)
    gathered = _sc_gather(queue2, (safe >> 1).reshape(-1))
    retrieved, w8, d8 = _run_rescore(
        query, gathered.reshape(_BATCH, _NCAND * 2 * _DIM), cols, par)
    return retrieved, d8[:, 0], w8[:, :_K]
